# baseline (speedup 1.0000x reference)
import functools

import jax
import jax.numpy as jnp
from jax import lax
from jax.experimental import pallas as pl
from jax.experimental.pallas import tpu as pltpu
from jax.experimental.pallas import tpu_sc as plsc

N_NODES = 10000
N_EDGES = 160000
NC = 2
NS = 16


_DEG_K = 40
_DEG_EPT = N_EDGES // (NC * NS)
_DEG_NCH = _DEG_EPT // _DEG_K


def _deg_call(col2d, ones_k, zeros_n):
    mesh = plsc.VectorSubcoreMesh(core_axis_name="c", subcore_axis_name="s")

    @functools.partial(
        pl.kernel,
        out_type=[jax.ShapeDtypeStruct((N_NODES,), jnp.float32)] * 2,
        mesh=mesh,
        scratch_types=[
            pltpu.VMEM((_DEG_NCH, _DEG_K), jnp.int32),
            pltpu.VMEM((_DEG_K,), jnp.float32),
            pltpu.VMEM_SHARED((N_NODES,), jnp.float32),
            pltpu.SemaphoreType.DMA,
        ],
    )
    def deg_kernel(col_hbm, ones_hbm, zeros_hbm, dega, degb,
                   idxb, ones_buf, acc, ssem):
        c = lax.axis_index("c")
        s = lax.axis_index("s")
        w = c * NS + s

        pltpu.sync_copy(col_hbm.at[w], idxb)
        pltpu.sync_copy(ones_hbm, ones_buf)

        @pl.when(s == 0)
        def _():
            pltpu.sync_copy(zeros_hbm, acc)

        plsc.subcore_barrier()

        def fire(i, carry):
            pltpu.async_copy(ones_buf, acc.at[idxb.at[i]], ssem, add=True)
            return carry

        lax.fori_loop(0, _DEG_NCH, fire, 0)

        def drain(i, carry):
            pltpu.make_async_copy(ones_buf, acc.at[idxb.at[0]], ssem).wait()
            return carry

        lax.fori_loop(0, _DEG_NCH, drain, 0)
        plsc.subcore_barrier()

        @pl.when(s == 0)
        def _():
            @pl.when(c == 0)
            def _():
                pltpu.sync_copy(acc, dega)

            @pl.when(c == 1)
            def _():
                pltpu.sync_copy(acc, degb)

    return deg_kernel(col2d, ones_k, zeros_n)


_NB = 3


def _make_agg(width, feature_split, k=128, tiled=True, nb=_NB, pd=2):
    epc = N_EDGES if feature_split else N_EDGES // NC
    ept = epc // NS
    nchm = (ept // k // nb) * nb
    if nchm * k == ept:
        nchm -= nb
    tail = ept - nchm * k
    assert 0 < tail <= nb * k and tail % 8 == 0 and nchm >= 2 * nb
    ntail_full, tail_rem = divmod(tail, k)
    rpt = 624
    rtail0 = NS * rpt
    rtailn = N_NODES - rtail0
    mesh = plsc.VectorSubcoreMesh(core_axis_name="c", subcore_axis_name="s")

    rem = tail_rem if tail_rem else k

    @functools.partial(
        pl.kernel,
        out_type=[jax.ShapeDtypeStruct((N_NODES, width), jnp.float32)] * 2,
        mesh=mesh,
        scratch_types=(
            [pltpu.VMEM((k,), jnp.int32)] * (2 * nb)
            + [pltpu.VMEM((rem,), jnp.int32)] * 2
            + [pltpu.VMEM((k, width), jnp.float32)] * nb
            + [pltpu.SemaphoreType.DMA] * (4 * nb + 3)
            + [pltpu.VMEM_SHARED((N_NODES, width), jnp.float32)]
        ),
        compiler_params=(None if tiled
                         else pltpu.CompilerParams(use_tc_tiling_on_sc=False)),
    )
    def agg(*refs):
        (y0, y1, row_hbm, col_hbm, z0, z1), refs = refs[:6], refs[6:]
        ribufs, refs = list(refs[:nb]), refs[nb:]
        cibufs, refs = list(refs[:nb]), refs[nb:]
        (tri, tci), refs = refs[:2], refs[2:]
        rbufs, refs = list(refs[:nb]), refs[nb:]
        rsems, refs = list(refs[:nb]), refs[nb:]
        csems, refs = list(refs[:nb]), refs[nb:]
        gsems, refs = list(refs[:nb]), refs[nb:]
        ssems, refs = list(refs[:nb]), refs[nb:]
        (mt, mi, mo, acc) = refs
        c = lax.axis_index("c")
        s = lax.axis_index("s")

        def run(ytab, zout, ebase):
            r0 = s * rpt
            pltpu.async_copy(ytab.at[pl.ds(r0, rpt)],
                             acc.at[pl.ds(r0, rpt)], mi)

            @pl.when(s == 0)
            def _():
                pltpu.async_copy(ytab.at[pl.ds(rtail0, rtailn)],
                                 acc.at[pl.ds(rtail0, rtailn)], mo)

            def idx_start(i, b):
                base = ebase + i * k
                pltpu.async_copy(row_hbm.at[pl.ds(base, k)], ribufs[b],
                                 rsems[b])
                pltpu.async_copy(col_hbm.at[pl.ds(base, k)], cibufs[b],
                                 csems[b])

            def idxrow_wait(b):
                pltpu.make_async_copy(row_hbm.at[pl.ds(0, k)], ribufs[b],
                                      rsems[b]).wait()

            def idxcol_wait(b):
                pltpu.make_async_copy(col_hbm.at[pl.ds(0, k)], cibufs[b],
                                      csems[b]).wait()

            def gather_start(b):
                pltpu.async_copy(ytab.at[ribufs[b]], rbufs[b], gsems[b])

            def gather_wait(b):
                pltpu.make_async_copy(ytab.at[ribufs[b]], rbufs[b],
                                      gsems[b]).wait()

            def scatter_start(b):
                pltpu.async_copy(rbufs[b], acc.at[cibufs[b]], ssems[b],
                                 add=True)

            def scatter_wait(b):
                pltpu.make_async_copy(rbufs[b], acc.at[cibufs[b]],
                                      ssems[b]).wait()

            for b in range(pd):
                idx_start(b, b)
            for b in range(pd - 1):
                idxrow_wait(b)
                gather_start(b)
            pltpu.make_async_copy(ytab.at[pl.ds(r0, rpt)],
                                  acc.at[pl.ds(r0, rpt)], mi).wait()

            @pl.when(s == 0)
            def _():
                pltpu.make_async_copy(ytab.at[pl.ds(rtail0, rtailn)],
                                      acc.at[pl.ds(rtail0, rtailn)],
                                      mo).wait()

            plsc.subcore_barrier()

            def group(g, carry):
                for b in range(nb):
                    i = g * nb + b
                    bg = (b + pd - 1) % nb
                    bp = (b + pd) % nb

                    @pl.when(i + pd < nchm)
                    def _(i=i, bp=bp):
                        @pl.when(i >= nb - pd)
                        def _():
                            scatter_wait(bp)

                        idx_start(i + pd, bp)

                    @pl.when(i + pd - 1 < nchm)
                    def _(bg=bg):
                        idxrow_wait(bg)
                        gather_start(bg)

                    gather_wait(b)
                    idxcol_wait(b)
                    scatter_start(b)
                return carry

            lax.fori_loop(0, nchm // nb, group, 0)

            for t in range(ntail_full):
                i = nchm + t
                b = i % nb
                scatter_wait(b)
                idx_start(i, b)
                idxrow_wait(b)
                gather_start(b)
                gather_wait(b)
                idxcol_wait(b)
                scatter_start(b)
            if tail_rem:
                rbase = ebase + (nchm + ntail_full) * k
                br = (nchm + ntail_full) % nb
                scatter_wait(br)
                pltpu.async_copy(row_hbm.at[pl.ds(rbase, tail_rem)], tri,
                                 rsems[br])
                pltpu.async_copy(col_hbm.at[pl.ds(rbase, tail_rem)], tci,
                                 csems[br])
                pltpu.make_async_copy(row_hbm.at[pl.ds(0, tail_rem)], tri,
                                      rsems[br]).wait()
                rslice = rbufs[br].at[pl.ds(0, tail_rem)]
                pltpu.async_copy(ytab.at[tri], rslice, mt)
                pltpu.make_async_copy(ytab.at[tri], rslice, mt).wait()
                pltpu.make_async_copy(col_hbm.at[pl.ds(0, tail_rem)], tci,
                                      csems[br]).wait()
                pltpu.async_copy(rslice, acc.at[tci], mt, add=True)
            for b in range(nb):
                if tail_rem and b == (nchm + ntail_full) % nb:
                    continue
                scatter_wait(b)
            if tail_rem:
                pltpu.make_async_copy(
                    rbufs[(nchm + ntail_full) % nb].at[pl.ds(0, tail_rem)],
                    acc.at[tci], mt).wait()
            plsc.subcore_barrier()

            pltpu.sync_copy(acc.at[pl.ds(r0, rpt)], zout.at[pl.ds(r0, rpt)])

            @pl.when(s == 0)
            def _():
                pltpu.sync_copy(acc.at[pl.ds(rtail0, rtailn)],
                                zout.at[pl.ds(rtail0, rtailn)])

        core_e0 = 0 if feature_split else epc

        @pl.when(c == 0)
        def _():
            run(y0, z0, s * ept)

        @pl.when(c == 1)
        def _():
            run(y1, z1, core_e0 + s * ept)

    return agg


_BN = 1000


def _mm1_call(x, w1):
    n, d = x.shape
    h = w1.shape[1]

    def body(x_ref, w_ref, xw_ref):
        xw_ref[...] = jnp.dot(x_ref[...].astype(jnp.bfloat16),
                              w_ref[...].astype(jnp.bfloat16),
                              preferred_element_type=jnp.float32)

    return pl.pallas_call(
        body,
        grid=(n // _BN,),
        in_specs=[
            pl.BlockSpec((_BN, d), lambda i: (i, 0)),
            pl.BlockSpec((d, h), lambda i: (0, 0)),
        ],
        out_specs=pl.BlockSpec((_BN, h), lambda i: (i, 0)),
        out_shape=jax.ShapeDtypeStruct((n, h), jnp.float32),
    )(x, w1)


def _scale1_call(xw, dega, degb):
    n, h = xw.shape
    hh = h // 2

    def body(xw_ref, da_ref, db_ref, ylo_ref, yhi_ref, dinv_ref):
        deg = da_ref[...] + db_ref[...] + 1.0
        dinv = lax.rsqrt(deg)
        y = xw_ref[...] * dinv
        ylo_ref[...] = y[:, :hh]
        yhi_ref[...] = y[:, hh:]
        dinv_ref[...] = dinv

    return pl.pallas_call(
        body,
        grid=(n // _BN,),
        in_specs=[
            pl.BlockSpec((_BN, h), lambda i: (i, 0)),
            pl.BlockSpec((_BN, 1), lambda i: (i, 0)),
            pl.BlockSpec((_BN, 1), lambda i: (i, 0)),
        ],
        out_specs=[
            pl.BlockSpec((_BN, hh), lambda i: (i, 0)),
            pl.BlockSpec((_BN, hh), lambda i: (i, 0)),
            pl.BlockSpec((_BN, 1), lambda i: (i, 0)),
        ],
        out_shape=[
            jax.ShapeDtypeStruct((n, hh), jnp.float32),
            jax.ShapeDtypeStruct((n, hh), jnp.float32),
            jax.ShapeDtypeStruct((n, 1), jnp.float32),
        ],
    )(xw, dega, degb)


def _dense2_call(zlo, zhi, dinv, b1, w2):
    n, hh = zlo.shape
    h = 2 * hh
    cdim = w2.shape[1]

    def body(zlo_ref, zhi_ref, dinv_ref, b1_ref, w_ref, y2_ref):
        z = jnp.concatenate([zlo_ref[...], zhi_ref[...]], axis=1)
        dinv = dinv_ref[...]
        hmat = jnp.maximum(z * dinv + b1_ref[...], 0.0)
        y2_ref[...] = jnp.dot(
            hmat, w_ref[...], preferred_element_type=jnp.float32) * dinv

    return pl.pallas_call(
        body,
        grid=(n // _BN,),
        in_specs=[
            pl.BlockSpec((_BN, hh), lambda i: (i, 0)),
            pl.BlockSpec((_BN, hh), lambda i: (i, 0)),
            pl.BlockSpec((_BN, 1), lambda i: (i, 0)),
            pl.BlockSpec((1, h), lambda i: (0, 0)),
            pl.BlockSpec((h, cdim), lambda i: (0, 0)),
        ],
        out_specs=pl.BlockSpec((_BN, cdim), lambda i: (i, 0)),
        out_shape=jax.ShapeDtypeStruct((n, cdim), jnp.float32),
    )(zlo, zhi, dinv, b1, w2)


def _out_call(z2a, z2b, y2p, dinv, b2):
    n = y2p.shape[0]
    cdim = b2.shape[1]

    def body(za_ref, zb_ref, y2_ref, dinv_ref, b2_ref, o_ref):
        acc = za_ref[...] + zb_ref[...] - y2_ref[...]
        o_ref[...] = acc * dinv_ref[...] + b2_ref[...]

    return pl.pallas_call(
        body,
        grid=(n // _BN,),
        in_specs=[
            pl.BlockSpec((_BN, cdim), lambda i: (i, 0)),
            pl.BlockSpec((_BN, cdim), lambda i: (i, 0)),
            pl.BlockSpec((_BN, cdim), lambda i: (i, 0)),
            pl.BlockSpec((_BN, 1), lambda i: (i, 0)),
            pl.BlockSpec((1, cdim), lambda i: (0, 0)),
        ],
        out_specs=pl.BlockSpec((_BN, cdim), lambda i: (i, 0)),
        out_shape=jax.ShapeDtypeStruct((n, cdim), jnp.float32),
    )(z2a, z2b, y2p, dinv, b2)


def kernel(x, edge_index, W1, b1, W2, b2):
    ei = edge_index.astype(jnp.int32)
    row, col = ei[0], ei[1]

    ones_k = jnp.ones((_DEG_K,), jnp.float32)
    zeros_n = jnp.zeros((N_NODES,), jnp.float32)
    col2d = col.reshape(NC * NS, _DEG_NCH, _DEG_K)
    dega, degb = _deg_call(col2d, ones_k, zeros_n)

    xw = _mm1_call(x, W1)
    ylo, yhi, dinv = _scale1_call(
        xw, dega.reshape(N_NODES, 1), degb.reshape(N_NODES, 1))

    agg1 = _make_agg(W1.shape[1] // 2, feature_split=True, k=64, nb=5, pd=3)
    zlo, zhi = agg1(ylo, yhi, row, col)

    y2p = _dense2_call(zlo, zhi, dinv, b1.reshape(1, -1), W2)

    agg2 = _make_agg(W2.shape[1], feature_split=False, tiled=False,
                     k=64, nb=5, pd=3)
    z2a, z2b = agg2(y2p, y2p, row, col)

    return _out_call(z2a, z2b, y2p, dinv, b2.reshape(1, -1))


# speedup vs baseline: 1.0131x; 1.0131x over previous
import functools

import jax
import jax.numpy as jnp
from jax import lax
from jax.experimental import pallas as pl
from jax.experimental.pallas import tpu as pltpu
from jax.experimental.pallas import tpu_sc as plsc

N_NODES = 10000
N_EDGES = 160000
NC = 2
NS = 16


_DEG_K = 40
_DEG_EPT = N_EDGES // (NC * NS)
_DEG_NCH = _DEG_EPT // _DEG_K


def _deg_call(col, ones_k, zeros_n):
    mesh = plsc.VectorSubcoreMesh(core_axis_name="c", subcore_axis_name="s")

    @functools.partial(
        pl.kernel,
        out_type=[jax.ShapeDtypeStruct((N_NODES,), jnp.float32)] * 2,
        mesh=mesh,
        scratch_types=[
            pltpu.VMEM((_DEG_EPT,), jnp.int32),
            pltpu.VMEM((_DEG_K,), jnp.float32),
            pltpu.VMEM_SHARED((N_NODES,), jnp.float32),
            pltpu.SemaphoreType.DMA,
        ],
    )
    def deg_kernel(col_hbm, ones_hbm, zeros_hbm, dega, degb,
                   idxb, ones_buf, acc, ssem):
        c = lax.axis_index("c")
        s = lax.axis_index("s")
        w = c * NS + s

        pltpu.sync_copy(col_hbm.at[pl.ds(w * _DEG_EPT, _DEG_EPT)], idxb)
        pltpu.sync_copy(ones_hbm, ones_buf)

        @pl.when(s == 0)
        def _():
            pltpu.sync_copy(zeros_hbm, acc)

        plsc.subcore_barrier()

        def fire(i, carry):
            pltpu.async_copy(ones_buf, acc.at[idxb.at[pl.ds(i * _DEG_K,
                                                            _DEG_K)]],
                             ssem, add=True)
            return carry

        lax.fori_loop(0, _DEG_NCH, fire, 0)

        def drain(i, carry):
            pltpu.make_async_copy(
                ones_buf, acc.at[idxb.at[pl.ds(0, _DEG_K)]], ssem).wait()
            return carry

        lax.fori_loop(0, _DEG_NCH, drain, 0)
        plsc.subcore_barrier()

        @pl.when(s == 0)
        def _():
            @pl.when(c == 0)
            def _():
                pltpu.sync_copy(acc, dega)

            @pl.when(c == 1)
            def _():
                pltpu.sync_copy(acc, degb)

    return deg_kernel(col, ones_k, zeros_n)


_NB = 3


def _make_agg(width, feature_split, k=128, tiled=True, nb=_NB, pd=2):
    epc = N_EDGES if feature_split else N_EDGES // NC
    ept = epc // NS
    nchm = (ept // k // nb) * nb
    if nchm * k == ept:
        nchm -= nb
    tail = ept - nchm * k
    assert 0 < tail <= nb * k and tail % 8 == 0 and nchm >= 2 * nb
    ntail_full, tail_rem = divmod(tail, k)
    rpt = 624
    rtail0 = NS * rpt
    rtailn = N_NODES - rtail0
    mesh = plsc.VectorSubcoreMesh(core_axis_name="c", subcore_axis_name="s")

    rem = tail_rem if tail_rem else k

    @functools.partial(
        pl.kernel,
        out_type=[jax.ShapeDtypeStruct((N_NODES, width), jnp.float32)] * 2,
        mesh=mesh,
        scratch_types=(
            [pltpu.VMEM((k,), jnp.int32)] * (2 * nb)
            + [pltpu.VMEM((rem,), jnp.int32)] * 2
            + [pltpu.VMEM((k, width), jnp.float32)] * nb
            + [pltpu.SemaphoreType.DMA] * (4 * nb + 3)
            + [pltpu.VMEM_SHARED((N_NODES, width), jnp.float32)]
        ),
        compiler_params=(None if tiled
                         else pltpu.CompilerParams(use_tc_tiling_on_sc=False)),
    )
    def agg(*refs):
        (y0, y1, row_hbm, col_hbm, z0, z1), refs = refs[:6], refs[6:]
        ribufs, refs = list(refs[:nb]), refs[nb:]
        cibufs, refs = list(refs[:nb]), refs[nb:]
        (tri, tci), refs = refs[:2], refs[2:]
        rbufs, refs = list(refs[:nb]), refs[nb:]
        rsems, refs = list(refs[:nb]), refs[nb:]
        csems, refs = list(refs[:nb]), refs[nb:]
        gsems, refs = list(refs[:nb]), refs[nb:]
        ssems, refs = list(refs[:nb]), refs[nb:]
        (mt, mi, mo, acc) = refs
        c = lax.axis_index("c")
        s = lax.axis_index("s")

        def run(ytab, zout, ebase):
            r0 = s * rpt
            pltpu.async_copy(ytab.at[pl.ds(r0, rpt)],
                             acc.at[pl.ds(r0, rpt)], mi)

            @pl.when(s == 0)
            def _():
                pltpu.async_copy(ytab.at[pl.ds(rtail0, rtailn)],
                                 acc.at[pl.ds(rtail0, rtailn)], mo)

            def idx_start(i, b):
                base = ebase + i * k
                pltpu.async_copy(row_hbm.at[pl.ds(base, k)], ribufs[b],
                                 rsems[b])
                pltpu.async_copy(col_hbm.at[pl.ds(base, k)], cibufs[b],
                                 csems[b])

            def idxrow_wait(b):
                pltpu.make_async_copy(row_hbm.at[pl.ds(0, k)], ribufs[b],
                                      rsems[b]).wait()

            def idxcol_wait(b):
                pltpu.make_async_copy(col_hbm.at[pl.ds(0, k)], cibufs[b],
                                      csems[b]).wait()

            def gather_start(b):
                pltpu.async_copy(ytab.at[ribufs[b]], rbufs[b], gsems[b])

            def gather_wait(b):
                pltpu.make_async_copy(ytab.at[ribufs[b]], rbufs[b],
                                      gsems[b]).wait()

            def scatter_start(b):
                pltpu.async_copy(rbufs[b], acc.at[cibufs[b]], ssems[b],
                                 add=True)

            def scatter_wait(b):
                pltpu.make_async_copy(rbufs[b], acc.at[cibufs[b]],
                                      ssems[b]).wait()

            for b in range(pd):
                idx_start(b, b)
            for b in range(pd - 1):
                idxrow_wait(b)
                gather_start(b)
            pltpu.make_async_copy(ytab.at[pl.ds(r0, rpt)],
                                  acc.at[pl.ds(r0, rpt)], mi).wait()

            @pl.when(s == 0)
            def _():
                pltpu.make_async_copy(ytab.at[pl.ds(rtail0, rtailn)],
                                      acc.at[pl.ds(rtail0, rtailn)],
                                      mo).wait()

            plsc.subcore_barrier()

            def group(g, carry):
                for b in range(nb):
                    i = g * nb + b
                    bg = (b + pd - 1) % nb
                    bp = (b + pd) % nb

                    @pl.when(i + pd < nchm)
                    def _(i=i, bp=bp):
                        @pl.when(i >= nb - pd)
                        def _():
                            scatter_wait(bp)

                        idx_start(i + pd, bp)

                    @pl.when(i + pd - 1 < nchm)
                    def _(bg=bg):
                        idxrow_wait(bg)
                        gather_start(bg)

                    gather_wait(b)
                    idxcol_wait(b)
                    scatter_start(b)
                return carry

            lax.fori_loop(0, nchm // nb, group, 0)

            for t in range(ntail_full):
                i = nchm + t
                b = i % nb
                scatter_wait(b)
                idx_start(i, b)
                idxrow_wait(b)
                gather_start(b)
                gather_wait(b)
                idxcol_wait(b)
                scatter_start(b)
            if tail_rem:
                rbase = ebase + (nchm + ntail_full) * k
                br = (nchm + ntail_full) % nb
                scatter_wait(br)
                pltpu.async_copy(row_hbm.at[pl.ds(rbase, tail_rem)], tri,
                                 rsems[br])
                pltpu.async_copy(col_hbm.at[pl.ds(rbase, tail_rem)], tci,
                                 csems[br])
                pltpu.make_async_copy(row_hbm.at[pl.ds(0, tail_rem)], tri,
                                      rsems[br]).wait()
                rslice = rbufs[br].at[pl.ds(0, tail_rem)]
                pltpu.async_copy(ytab.at[tri], rslice, mt)
                pltpu.make_async_copy(ytab.at[tri], rslice, mt).wait()
                pltpu.make_async_copy(col_hbm.at[pl.ds(0, tail_rem)], tci,
                                      csems[br]).wait()
                pltpu.async_copy(rslice, acc.at[tci], mt, add=True)
            for b in range(nb):
                if tail_rem and b == (nchm + ntail_full) % nb:
                    continue
                scatter_wait(b)
            if tail_rem:
                pltpu.make_async_copy(
                    rbufs[(nchm + ntail_full) % nb].at[pl.ds(0, tail_rem)],
                    acc.at[tci], mt).wait()
            plsc.subcore_barrier()

            pltpu.sync_copy(acc.at[pl.ds(r0, rpt)], zout.at[pl.ds(r0, rpt)])

            @pl.when(s == 0)
            def _():
                pltpu.sync_copy(acc.at[pl.ds(rtail0, rtailn)],
                                zout.at[pl.ds(rtail0, rtailn)])

        core_e0 = 0 if feature_split else epc

        @pl.when(c == 0)
        def _():
            run(y0, z0, s * ept)

        @pl.when(c == 1)
        def _():
            run(y1, z1, core_e0 + s * ept)

    return agg


_BN = 1000


def _mm1_call(x, w1):
    n, d = x.shape
    h = w1.shape[1]

    def body(x_ref, w_ref, xw_ref):
        xw_ref[...] = jnp.dot(x_ref[...].astype(jnp.bfloat16),
                              w_ref[...].astype(jnp.bfloat16),
                              preferred_element_type=jnp.float32)

    return pl.pallas_call(
        body,
        grid=(n // _BN,),
        in_specs=[
            pl.BlockSpec((_BN, d), lambda i: (i, 0)),
            pl.BlockSpec((d, h), lambda i: (0, 0)),
        ],
        out_specs=pl.BlockSpec((_BN, h), lambda i: (i, 0)),
        out_shape=jax.ShapeDtypeStruct((n, h), jnp.float32),
    )(x, w1)


def _scale1_call(xw, dega, degb):
    n, h = xw.shape
    hh = h // 2

    def body(xw_ref, da_ref, db_ref, ylo_ref, yhi_ref, dinv_ref):
        deg = da_ref[...] + db_ref[...] + 1.0
        dinv = lax.rsqrt(deg)
        y = xw_ref[...] * dinv
        ylo_ref[...] = y[:, :hh]
        yhi_ref[...] = y[:, hh:]
        dinv_ref[...] = dinv

    return pl.pallas_call(
        body,
        grid=(n // _BN,),
        in_specs=[
            pl.BlockSpec((_BN, h), lambda i: (i, 0)),
            pl.BlockSpec((_BN, 1), lambda i: (i, 0)),
            pl.BlockSpec((_BN, 1), lambda i: (i, 0)),
        ],
        out_specs=[
            pl.BlockSpec((_BN, hh), lambda i: (i, 0)),
            pl.BlockSpec((_BN, hh), lambda i: (i, 0)),
            pl.BlockSpec((_BN, 1), lambda i: (i, 0)),
        ],
        out_shape=[
            jax.ShapeDtypeStruct((n, hh), jnp.float32),
            jax.ShapeDtypeStruct((n, hh), jnp.float32),
            jax.ShapeDtypeStruct((n, 1), jnp.float32),
        ],
    )(xw, dega, degb)


def _dense2_call(zlo, zhi, dinv, b1, w2):
    n, hh = zlo.shape
    h = 2 * hh
    cdim = w2.shape[1]

    def body(zlo_ref, zhi_ref, dinv_ref, b1_ref, w_ref, y2_ref):
        z = jnp.concatenate([zlo_ref[...], zhi_ref[...]], axis=1)
        dinv = dinv_ref[...]
        hmat = jnp.maximum(z * dinv + b1_ref[...], 0.0)
        y2_ref[...] = jnp.dot(
            hmat, w_ref[...], preferred_element_type=jnp.float32) * dinv

    return pl.pallas_call(
        body,
        grid=(n // _BN,),
        in_specs=[
            pl.BlockSpec((_BN, hh), lambda i: (i, 0)),
            pl.BlockSpec((_BN, hh), lambda i: (i, 0)),
            pl.BlockSpec((_BN, 1), lambda i: (i, 0)),
            pl.BlockSpec((1, h), lambda i: (0, 0)),
            pl.BlockSpec((h, cdim), lambda i: (0, 0)),
        ],
        out_specs=pl.BlockSpec((_BN, cdim), lambda i: (i, 0)),
        out_shape=jax.ShapeDtypeStruct((n, cdim), jnp.float32),
    )(zlo, zhi, dinv, b1, w2)


def _out_call(z2a, z2b, y2p, dinv, b2):
    n = y2p.shape[0]
    cdim = b2.shape[1]

    def body(za_ref, zb_ref, y2_ref, dinv_ref, b2_ref, o_ref):
        acc = za_ref[...] + zb_ref[...] - y2_ref[...]
        o_ref[...] = acc * dinv_ref[...] + b2_ref[...]

    return pl.pallas_call(
        body,
        grid=(n // _BN,),
        in_specs=[
            pl.BlockSpec((_BN, cdim), lambda i: (i, 0)),
            pl.BlockSpec((_BN, cdim), lambda i: (i, 0)),
            pl.BlockSpec((_BN, cdim), lambda i: (i, 0)),
            pl.BlockSpec((_BN, 1), lambda i: (i, 0)),
            pl.BlockSpec((1, cdim), lambda i: (0, 0)),
        ],
        out_specs=pl.BlockSpec((_BN, cdim), lambda i: (i, 0)),
        out_shape=jax.ShapeDtypeStruct((n, cdim), jnp.float32),
    )(z2a, z2b, y2p, dinv, b2)


def kernel(x, edge_index, W1, b1, W2, b2):
    ei = edge_index.astype(jnp.int32)
    row, col = ei[0], ei[1]

    ones_k = jnp.ones((_DEG_K,), jnp.float32)
    zeros_n = jnp.zeros((N_NODES,), jnp.float32)
    dega, degb = _deg_call(col, ones_k, zeros_n)

    xw = _mm1_call(x, W1)
    ylo, yhi, dinv = _scale1_call(
        xw, dega.reshape(N_NODES, 1), degb.reshape(N_NODES, 1))

    agg1 = _make_agg(W1.shape[1] // 2, feature_split=True, k=80, nb=4)
    zlo, zhi = agg1(ylo, yhi, row, col)

    y2p = _dense2_call(zlo, zhi, dinv, b1.reshape(1, -1), W2)

    agg2 = _make_agg(W2.shape[1], feature_split=False, tiled=False)
    z2a, z2b = agg2(y2p, y2p, row, col)

    return _out_call(z2a, z2b, y2p, dinv, b2.reshape(1, -1))


# speedup vs baseline: 1.0262x; 1.0130x over previous
import functools

import jax
import jax.numpy as jnp
from jax import lax
from jax.experimental import pallas as pl
from jax.experimental.pallas import tpu as pltpu
from jax.experimental.pallas import tpu_sc as plsc

N_NODES = 10000
N_EDGES = 160000
NC = 2
NS = 16


_DEG_K = 40
_DEG_EPT = N_EDGES // (NC * NS)
_DEG_NCH = _DEG_EPT // _DEG_K


def _deg_call(col, ones_k, zeros_n):
    mesh = plsc.VectorSubcoreMesh(core_axis_name="c", subcore_axis_name="s")

    @functools.partial(
        pl.kernel,
        out_type=[jax.ShapeDtypeStruct((N_NODES,), jnp.float32)] * 2,
        mesh=mesh,
        scratch_types=[
            pltpu.VMEM((_DEG_EPT,), jnp.int32),
            pltpu.VMEM((_DEG_K,), jnp.float32),
            pltpu.VMEM_SHARED((N_NODES,), jnp.float32),
            pltpu.SemaphoreType.DMA,
        ],
    )
    def deg_kernel(col_hbm, ones_hbm, zeros_hbm, dega, degb,
                   idxb, ones_buf, acc, ssem):
        c = lax.axis_index("c")
        s = lax.axis_index("s")
        w = c * NS + s

        pltpu.sync_copy(col_hbm.at[pl.ds(w * _DEG_EPT, _DEG_EPT)], idxb)
        pltpu.sync_copy(ones_hbm, ones_buf)

        @pl.when(s == 0)
        def _():
            pltpu.sync_copy(zeros_hbm, acc)

        plsc.subcore_barrier()

        def fire(i, carry):
            pltpu.async_copy(ones_buf, acc.at[idxb.at[pl.ds(i * _DEG_K,
                                                            _DEG_K)]],
                             ssem, add=True)
            return carry

        lax.fori_loop(0, _DEG_NCH, fire, 0)

        def drain(i, carry):
            pltpu.make_async_copy(
                ones_buf, acc.at[idxb.at[pl.ds(0, _DEG_K)]], ssem).wait()
            return carry

        lax.fori_loop(0, _DEG_NCH, drain, 0)
        plsc.subcore_barrier()

        @pl.when(s == 0)
        def _():
            @pl.when(c == 0)
            def _():
                pltpu.sync_copy(acc, dega)

            @pl.when(c == 1)
            def _():
                pltpu.sync_copy(acc, degb)

    return deg_kernel(col, ones_k, zeros_n)


_NB = 3


def _make_agg(width, feature_split, k=128, tiled=True, nb=_NB, pd=2):
    epc = N_EDGES if feature_split else N_EDGES // NC
    ept = epc // NS
    nchm = (ept // k // nb) * nb
    if nchm * k == ept:
        nchm -= nb
    tail = ept - nchm * k
    assert 0 < tail <= nb * k and tail % 8 == 0 and nchm >= 2 * nb
    ntail_full, tail_rem = divmod(tail, k)
    rpt = 624
    rtail0 = NS * rpt
    rtailn = N_NODES - rtail0
    mesh = plsc.VectorSubcoreMesh(core_axis_name="c", subcore_axis_name="s")

    rem = tail_rem if tail_rem else k

    @functools.partial(
        pl.kernel,
        out_type=[jax.ShapeDtypeStruct((N_NODES, width), jnp.float32)] * 2,
        mesh=mesh,
        scratch_types=(
            [pltpu.VMEM((k,), jnp.int32)] * (2 * nb)
            + [pltpu.VMEM((rem,), jnp.int32)] * 2
            + [pltpu.VMEM((k, width), jnp.float32)] * nb
            + [pltpu.SemaphoreType.DMA] * (4 * nb + 3)
            + [pltpu.VMEM_SHARED((N_NODES, width), jnp.float32)]
        ),
        compiler_params=(None if tiled
                         else pltpu.CompilerParams(use_tc_tiling_on_sc=False)),
    )
    def agg(*refs):
        (y0, y1, row_hbm, col_hbm, z0, z1), refs = refs[:6], refs[6:]
        ribufs, refs = list(refs[:nb]), refs[nb:]
        cibufs, refs = list(refs[:nb]), refs[nb:]
        (tri, tci), refs = refs[:2], refs[2:]
        rbufs, refs = list(refs[:nb]), refs[nb:]
        rsems, refs = list(refs[:nb]), refs[nb:]
        csems, refs = list(refs[:nb]), refs[nb:]
        gsems, refs = list(refs[:nb]), refs[nb:]
        ssems, refs = list(refs[:nb]), refs[nb:]
        (mt, mi, mo, acc) = refs
        c = lax.axis_index("c")
        s = lax.axis_index("s")

        def run(ytab, zout, ebase):
            r0 = s * rpt
            pltpu.async_copy(ytab.at[pl.ds(r0, rpt)],
                             acc.at[pl.ds(r0, rpt)], mi)

            @pl.when(s == 0)
            def _():
                pltpu.async_copy(ytab.at[pl.ds(rtail0, rtailn)],
                                 acc.at[pl.ds(rtail0, rtailn)], mo)

            def idx_start(i, b):
                base = ebase + i * k
                pltpu.async_copy(row_hbm.at[pl.ds(base, k)], ribufs[b],
                                 rsems[b])
                pltpu.async_copy(col_hbm.at[pl.ds(base, k)], cibufs[b],
                                 csems[b])

            def idxrow_wait(b):
                pltpu.make_async_copy(row_hbm.at[pl.ds(0, k)], ribufs[b],
                                      rsems[b]).wait()

            def idxcol_wait(b):
                pltpu.make_async_copy(col_hbm.at[pl.ds(0, k)], cibufs[b],
                                      csems[b]).wait()

            def gather_start(b):
                pltpu.async_copy(ytab.at[ribufs[b]], rbufs[b], gsems[b])

            def gather_wait(b):
                pltpu.make_async_copy(ytab.at[ribufs[b]], rbufs[b],
                                      gsems[b]).wait()

            def scatter_start(b):
                pltpu.async_copy(rbufs[b], acc.at[cibufs[b]], ssems[b],
                                 add=True)

            def scatter_wait(b):
                pltpu.make_async_copy(rbufs[b], acc.at[cibufs[b]],
                                      ssems[b]).wait()

            for b in range(pd):
                idx_start(b, b)
            for b in range(pd - 1):
                idxrow_wait(b)
                gather_start(b)
            pltpu.make_async_copy(ytab.at[pl.ds(r0, rpt)],
                                  acc.at[pl.ds(r0, rpt)], mi).wait()

            @pl.when(s == 0)
            def _():
                pltpu.make_async_copy(ytab.at[pl.ds(rtail0, rtailn)],
                                      acc.at[pl.ds(rtail0, rtailn)],
                                      mo).wait()

            plsc.subcore_barrier()

            def group(g, carry):
                for b in range(nb):
                    i = g * nb + b
                    bg = (b + pd - 1) % nb
                    bp = (b + pd) % nb

                    @pl.when(i + pd < nchm)
                    def _(i=i, bp=bp):
                        @pl.when(i >= nb - pd)
                        def _():
                            scatter_wait(bp)

                        idx_start(i + pd, bp)

                    @pl.when(i + pd - 1 < nchm)
                    def _(bg=bg):
                        idxrow_wait(bg)
                        gather_start(bg)

                    gather_wait(b)
                    idxcol_wait(b)
                    scatter_start(b)
                return carry

            lax.fori_loop(0, nchm // nb, group, 0)

            for t in range(ntail_full):
                i = nchm + t
                b = i % nb
                scatter_wait(b)
                idx_start(i, b)
                idxrow_wait(b)
                gather_start(b)
                gather_wait(b)
                idxcol_wait(b)
                scatter_start(b)
            if tail_rem:
                rbase = ebase + (nchm + ntail_full) * k
                br = (nchm + ntail_full) % nb
                scatter_wait(br)
                pltpu.async_copy(row_hbm.at[pl.ds(rbase, tail_rem)], tri,
                                 rsems[br])
                pltpu.async_copy(col_hbm.at[pl.ds(rbase, tail_rem)], tci,
                                 csems[br])
                pltpu.make_async_copy(row_hbm.at[pl.ds(0, tail_rem)], tri,
                                      rsems[br]).wait()
                rslice = rbufs[br].at[pl.ds(0, tail_rem)]
                pltpu.async_copy(ytab.at[tri], rslice, mt)
                pltpu.make_async_copy(ytab.at[tri], rslice, mt).wait()
                pltpu.make_async_copy(col_hbm.at[pl.ds(0, tail_rem)], tci,
                                      csems[br]).wait()
                pltpu.async_copy(rslice, acc.at[tci], mt, add=True)
            for b in range(nb):
                if tail_rem and b == (nchm + ntail_full) % nb:
                    continue
                scatter_wait(b)
            if tail_rem:
                pltpu.make_async_copy(
                    rbufs[(nchm + ntail_full) % nb].at[pl.ds(0, tail_rem)],
                    acc.at[tci], mt).wait()
            plsc.subcore_barrier()

            pltpu.sync_copy(acc.at[pl.ds(r0, rpt)], zout.at[pl.ds(r0, rpt)])

            @pl.when(s == 0)
            def _():
                pltpu.sync_copy(acc.at[pl.ds(rtail0, rtailn)],
                                zout.at[pl.ds(rtail0, rtailn)])

        core_e0 = 0 if feature_split else epc

        @pl.when(c == 0)
        def _():
            run(y0, z0, s * ept)

        @pl.when(c == 1)
        def _():
            run(y1, z1, core_e0 + s * ept)

    return agg


_BN = 1000


def _mm1_call(x, w1):
    n, d = x.shape
    h = w1.shape[1]

    def body(x_ref, w_ref, xw_ref):
        xw_ref[...] = jnp.dot(x_ref[...].astype(jnp.bfloat16),
                              w_ref[...].astype(jnp.bfloat16),
                              preferred_element_type=jnp.float32)

    return pl.pallas_call(
        body,
        grid=(n // _BN,),
        in_specs=[
            pl.BlockSpec((_BN, d), lambda i: (i, 0)),
            pl.BlockSpec((d, h), lambda i: (0, 0)),
        ],
        out_specs=pl.BlockSpec((_BN, h), lambda i: (i, 0)),
        out_shape=jax.ShapeDtypeStruct((n, h), jnp.float32),
    )(x, w1)


def _scale1_call(xw, dega, degb):
    n, h = xw.shape
    hh = h // 2

    def body(xw_ref, da_ref, db_ref, ylo_ref, yhi_ref, dinv_ref):
        deg = da_ref[...] + db_ref[...] + 1.0
        dinv = lax.rsqrt(deg)
        y = xw_ref[...] * dinv
        ylo_ref[...] = y[:, :hh]
        yhi_ref[...] = y[:, hh:]
        dinv_ref[...] = dinv

    return pl.pallas_call(
        body,
        grid=(n // _BN,),
        in_specs=[
            pl.BlockSpec((_BN, h), lambda i: (i, 0)),
            pl.BlockSpec((_BN, 1), lambda i: (i, 0)),
            pl.BlockSpec((_BN, 1), lambda i: (i, 0)),
        ],
        out_specs=[
            pl.BlockSpec((_BN, hh), lambda i: (i, 0)),
            pl.BlockSpec((_BN, hh), lambda i: (i, 0)),
            pl.BlockSpec((_BN, 1), lambda i: (i, 0)),
        ],
        out_shape=[
            jax.ShapeDtypeStruct((n, hh), jnp.float32),
            jax.ShapeDtypeStruct((n, hh), jnp.float32),
            jax.ShapeDtypeStruct((n, 1), jnp.float32),
        ],
    )(xw, dega, degb)


def _dense2_call(zlo, zhi, dinv, b1, w2):
    n, hh = zlo.shape
    h = 2 * hh
    cdim = w2.shape[1]

    def body(zlo_ref, zhi_ref, dinv_ref, b1_ref, w_ref, y2_ref):
        z = jnp.concatenate([zlo_ref[...], zhi_ref[...]], axis=1)
        dinv = dinv_ref[...]
        hmat = jnp.maximum(z * dinv + b1_ref[...], 0.0)
        y2_ref[...] = jnp.dot(
            hmat, w_ref[...], preferred_element_type=jnp.float32) * dinv

    return pl.pallas_call(
        body,
        grid=(n // _BN,),
        in_specs=[
            pl.BlockSpec((_BN, hh), lambda i: (i, 0)),
            pl.BlockSpec((_BN, hh), lambda i: (i, 0)),
            pl.BlockSpec((_BN, 1), lambda i: (i, 0)),
            pl.BlockSpec((1, h), lambda i: (0, 0)),
            pl.BlockSpec((h, cdim), lambda i: (0, 0)),
        ],
        out_specs=pl.BlockSpec((_BN, cdim), lambda i: (i, 0)),
        out_shape=jax.ShapeDtypeStruct((n, cdim), jnp.float32),
    )(zlo, zhi, dinv, b1, w2)


def _out_call(z2a, z2b, y2p, dinv, b2):
    n = y2p.shape[0]
    cdim = b2.shape[1]

    def body(za_ref, zb_ref, y2_ref, dinv_ref, b2_ref, o_ref):
        acc = za_ref[...] + zb_ref[...] - y2_ref[...]
        o_ref[...] = acc * dinv_ref[...] + b2_ref[...]

    return pl.pallas_call(
        body,
        grid=(n // _BN,),
        in_specs=[
            pl.BlockSpec((_BN, cdim), lambda i: (i, 0)),
            pl.BlockSpec((_BN, cdim), lambda i: (i, 0)),
            pl.BlockSpec((_BN, cdim), lambda i: (i, 0)),
            pl.BlockSpec((_BN, 1), lambda i: (i, 0)),
            pl.BlockSpec((1, cdim), lambda i: (0, 0)),
        ],
        out_specs=pl.BlockSpec((_BN, cdim), lambda i: (i, 0)),
        out_shape=jax.ShapeDtypeStruct((n, cdim), jnp.float32),
    )(z2a, z2b, y2p, dinv, b2)


def kernel(x, edge_index, W1, b1, W2, b2):
    ei = edge_index.astype(jnp.int32)
    row, col = ei[0], ei[1]

    ones_k = jnp.ones((_DEG_K,), jnp.float32)
    zeros_n = jnp.zeros((N_NODES,), jnp.float32)
    dega, degb = _deg_call(col, ones_k, zeros_n)

    xw = _mm1_call(x, W1)
    ylo, yhi, dinv = _scale1_call(
        xw, dega.reshape(N_NODES, 1), degb.reshape(N_NODES, 1))

    agg1 = _make_agg(W1.shape[1] // 2, feature_split=True, k=96, nb=4)
    zlo, zhi = agg1(ylo, yhi, row, col)

    y2p = _dense2_call(zlo, zhi, dinv, b1.reshape(1, -1), W2)

    agg2 = _make_agg(W2.shape[1], feature_split=False, tiled=False)
    z2a, z2b = agg2(y2p, y2p, row, col)

    return _out_call(z2a, z2b, y2p, dinv, b2.reshape(1, -1))
